# baseline (speedup 1.0000x reference)
import functools

import jax
import jax.numpy as jnp
from jax.experimental import pallas as pl
from jax.experimental.pallas import tpu as pltpu

_NUM_CLASSES = 4
_IMG_SIZE = 640.0
_EPS = 1e-07
_B = 64
_T = 50
_TNEG = 500


def _giou(px, py, pw, ph, tx, ty, tw, th):
    x1n, y1n, x1x, y1x = px - pw / 2, py - ph / 2, px + pw / 2, py + ph / 2
    x2n, y2n, x2x, y2x = tx - tw / 2, ty - th / 2, tx + tw / 2, ty + th / 2
    iw = jnp.clip(jnp.minimum(x1x, x2x) - jnp.maximum(x1n, x2n), 0.0)
    ih = jnp.clip(jnp.minimum(y1x, y2x) - jnp.maximum(y1n, y2n), 0.0)
    inter = iw * ih
    union = pw * ph + tw * th - inter + _EPS
    iou = inter / union
    ew = jnp.maximum(x1x, x2x) - jnp.minimum(x1n, x2n)
    eh = jnp.maximum(y1x, y2x) - jnp.minimum(y1n, y2n)
    enclose = ew * eh + _EPS
    return iou - (enclose - union) / enclose


def _yolo_kernel(tgt_ref, anc_ref, p3_ref, p45_ref, nrc_ref,
                 out_ref, acc_ref):
    f32 = jnp.float32
    bf16 = jnp.bfloat16
    b = pl.program_id(0)
    tt = tgt_ref[0].T
    cls_id = tt[0:1, :].astype(jnp.int32)
    tx, ty, tw, th = tt[1:2, :], tt[2:3, :], tt[3:4, :], tt[4:5, :]
    valid = ((tx >= 0) & (tx <= 1) & (ty >= 0) & (ty <= 1) &
             (tw > 0) & (tw <= 1) & (th > 0) & (th <= 1))
    mask = (valid & (cls_id < _NUM_CLASSES)).astype(f32)
    n_s = jnp.sum(valid.astype(f32)) * 3.0

    def _div(j, w):
        return jnp.floor((j.astype(f32) + 0.5) * (1.0 / w)).astype(jnp.int32)

    nj = nrc_ref[0]

    j3 = nj[0:1, :]
    nr3 = _div(j3, 80)
    nc3 = j3 - nr3 * 80

    def one_scale(pred, h, w, a0, nrow, ncol):
        aw = [anc_ref[a0 + a, 0] / _IMG_SIZE for a in range(3)]
        ah = [anc_ref[a0 + a, 1] / _IMG_SIZE for a in range(3)]

        def ratio(a):
            inter = jnp.minimum(tw, aw[a]) * jnp.minimum(th, ah[a])
            union = tw * th + aw[a] * ah[a] - inter
            return inter / (union + _EPS)
        r0, r1, r2 = ratio(0), ratio(1), ratio(2)
        best = jnp.where(r1 > r0, 1, 0)
        best = jnp.where(r2 > jnp.maximum(r0, r1), 2, best)

        gxi = jnp.clip(jnp.floor(tx * w).astype(jnp.int32), 0, w - 1)
        gyi = jnp.clip(jnp.floor(ty * h).astype(jnp.int32), 0, h - 1)

        p_hi = pred.astype(bf16)
        p_lo = (pred - p_hi.astype(f32)).astype(bf16)
        phl = jnp.concatenate([p_hi, p_lo], axis=0)
        p2 = phl.reshape(54 * h, w)
        ohx = (jax.lax.broadcasted_iota(jnp.int32, (w, _T), 0)
               == gxi).astype(bf16)
        fx = jax.lax.dot(p2, ohx, preferred_element_type=f32)
        fsum = fx[0:27 * h] + fx[27 * h:]
        fa = jnp.where(best == 0, fsum[0:9 * h],
                       jnp.where(best == 1, fsum[9 * h:18 * h],
                                 fsum[18 * h:27 * h]))
        f9 = fa.reshape(9, h, _T)
        ohy = (jax.lax.broadcasted_iota(jnp.int32, (1, h, _T), 1)
               == gyi[:, None, :]).astype(f32)
        z = jnp.sum(f9 * ohy, axis=1)

        aw_sel = jnp.where(best == 0, aw[0], jnp.where(best == 1, aw[1], aw[2]))
        ah_sel = jnp.where(best == 0, ah[0], jnp.where(best == 1, ah[1], ah[2]))

        px = jax.nn.sigmoid(z[0:1]) + gxi.astype(f32) / w
        py = jax.nn.sigmoid(z[1:2]) + gyi.astype(f32) / h
        pw_ = jnp.exp(z[2:3]) * aw_sel
        ph_ = jnp.exp(z[3:4]) * ah_sel
        giou = _giou(px, py, pw_, ph_, tx, ty, tw, th)

        box_s = jnp.sum(mask * (1.0 - giou))
        zcl = z[5:9]
        ohc = (jax.lax.broadcasted_iota(jnp.int32, (_NUM_CLASSES, _T), 0)
               == cls_id).astype(f32)
        bce = jnp.sum(jax.nn.softplus(zcl) - ohc * zcl, axis=0,
                      keepdims=True) / _NUM_CLASSES
        cls_s = jnp.sum(mask * bce)
        zc = z[4:5]
        conf_t = jnp.clip(giou, 0.0, 1.0)
        conf_s = jnp.sum(mask * (jax.nn.softplus(zc) - conf_t * zc))

        sp = jax.nn.softplus(
            jnp.concatenate([pred[4], pred[13], pred[22]], axis=0))
        ohr = (jax.lax.broadcasted_iota(jnp.int32, (3 * h, _TNEG), 0)
               == nrow).astype(bf16)
        ohcol = (jax.lax.broadcasted_iota(jnp.int32, (128, _TNEG), 0)
                 == ncol).astype(bf16)
        counts = jax.lax.dot_general(ohr, ohcol, (((1,), (1,)), ((), ())),
                                     preferred_element_type=f32)
        neg_s = jnp.sum(sp * counts[:, 0:w])
        return box_s, cls_s, conf_s, neg_s

    def flat_scale(pred, h, w, a0, nrow, ncol, nrows):
        hw = h * w
        aw = [anc_ref[a0 + a, 0] / _IMG_SIZE for a in range(3)]
        ah = [anc_ref[a0 + a, 1] / _IMG_SIZE for a in range(3)]

        def ratio(a):
            inter = jnp.minimum(tw, aw[a]) * jnp.minimum(th, ah[a])
            union = tw * th + aw[a] * ah[a] - inter
            return inter / (union + _EPS)
        r0, r1, r2 = ratio(0), ratio(1), ratio(2)
        best = jnp.where(r1 > r0, 1, 0)
        best = jnp.where(r2 > jnp.maximum(r0, r1), 2, best)

        gxi = jnp.clip(jnp.floor(tx * w).astype(jnp.int32), 0, w - 1)
        gyi = jnp.clip(jnp.floor(ty * h).astype(jnp.int32), 0, h - 1)
        s = gyi * w + gxi

        p_hi = pred.astype(bf16)
        p_lo = (pred - p_hi.astype(f32)).astype(bf16)
        phl = jnp.concatenate([p_hi, p_lo], axis=0)
        ohs = (jax.lax.broadcasted_iota(jnp.int32, (hw, _T), 0)
               == s).astype(bf16)
        fx = jax.lax.dot(phl, ohs, preferred_element_type=f32)
        fsum = fx[0:27] + fx[27:]
        z = jnp.where(best == 0, fsum[0:9],
                      jnp.where(best == 1, fsum[9:18],
                                fsum[18:27]))

        aw_sel = jnp.where(best == 0, aw[0], jnp.where(best == 1, aw[1], aw[2]))
        ah_sel = jnp.where(best == 0, ah[0], jnp.where(best == 1, ah[1], ah[2]))
        px = jax.nn.sigmoid(z[0:1]) + gxi.astype(f32) / w
        py = jax.nn.sigmoid(z[1:2]) + gyi.astype(f32) / h
        pw_ = jnp.exp(z[2:3]) * aw_sel
        ph_ = jnp.exp(z[3:4]) * ah_sel
        giou = _giou(px, py, pw_, ph_, tx, ty, tw, th)

        box_s = jnp.sum(mask * (1.0 - giou))
        zcl = z[5:9]
        ohc = (jax.lax.broadcasted_iota(jnp.int32, (_NUM_CLASSES, _T), 0)
               == cls_id).astype(f32)
        bce = jnp.sum(jax.nn.softplus(zcl) - ohc * zcl, axis=0,
                      keepdims=True) / _NUM_CLASSES
        cls_s = jnp.sum(mask * bce)
        zc = z[4:5]
        conf_t = jnp.clip(giou, 0.0, 1.0)
        conf_s = jnp.sum(mask * (jax.nn.softplus(zc) - conf_t * zc))

        hwpad = ((hw + 127) // 128) * 128
        sp0 = jax.nn.softplus(
            jnp.concatenate([pred[4:5], pred[13:14], pred[22:23]], axis=0))
        if hwpad > hw:
            sp0 = jnp.concatenate(
                [sp0, jnp.zeros((3, hwpad - hw), f32)], axis=1)
        sp = sp0.reshape(nrows, 128)
        ohr = (jax.lax.broadcasted_iota(jnp.int32, (nrows, _TNEG), 0)
               == nrow).astype(bf16)
        ohcol = (jax.lax.broadcasted_iota(jnp.int32, (128, _TNEG), 0)
                 == ncol).astype(bf16)
        counts = jax.lax.dot_general(ohr, ohcol, (((1,), (1,)), ((), ())),
                                     preferred_element_type=f32)
        neg_s = jnp.sum(sp * counts)
        return box_s, cls_s, conf_s, neg_s

    def padded_rc(si, hw):
        hwpad = ((hw + 127) // 128) * 128
        j = nj[si:si + 1, :]
        a = (j >= hw).astype(jnp.int32) + (j >= 2 * hw).astype(jnp.int32)
        jp = a * hwpad + (j - a * hw)
        return jp // 128, jp - (jp // 128) * 128

    nr4, nc4 = padded_rc(1, 1600)
    nr5, nc5 = padded_rc(2, 400)
    b3 = one_scale(p3_ref[0], 80, 80, 0, nr3, nc3)
    p45 = p45_ref[0]
    b4 = flat_scale(p45[:, 0:1600], 40, 40, 3, nr4, nc4, 39)
    b5 = flat_scale(p45[:, 1600:2000], 20, 20, 6, nr5, nc5, 12)
    box_s = b3[0] + b4[0] + b5[0]
    cls_s = b3[1] + b4[1] + b5[1]
    conf_s = b3[2] + b4[2] + b5[2]
    neg_s = b3[3] + b4[3] + b5[3]

    lanes = jax.lax.broadcasted_iota(jnp.int32, (1, 128), 1)
    vec = (jnp.where(lanes == 0, box_s, 0.0) +
           jnp.where(lanes == 1, cls_s, 0.0) +
           jnp.where(lanes == 2, conf_s, 0.0) +
           jnp.where(lanes == 3, n_s, 0.0) +
           jnp.where(lanes == 4, neg_s, 0.0))

    @pl.when(b == 0)
    def _():
        acc_ref[...] = vec

    @pl.when(b > 0)
    def _():
        acc_ref[...] += vec

    @pl.when(b == _B - 1)
    def _():
        v = acc_ref[...]

        def lane(k):
            return jnp.sum(jnp.where(lanes == k, v, 0.0))
        box, cls_, conf = lane(0), lane(1), lane(2)
        n, neg = lane(3), lane(4)
        denom = jnp.maximum(n, 1.0)
        pos = n > 0
        box = jnp.where(pos, box / denom, box)
        cls_ = jnp.where(pos, cls_ / denom, cls_)
        conf = jnp.where(pos, conf / denom, conf)
        conf = conf + neg * (0.5 / (_TNEG * float(_B * _TNEG)))
        total = box + cls_ + conf
        out_ref[...] = (jnp.where(lanes == 0, total, 0.0) +
                        jnp.where(lanes == 1, box, 0.0) +
                        jnp.where(lanes == 2, cls_, 0.0) +
                        jnp.where(lanes == 3, conf, 0.0))


@functools.partial(jax.jit, static_argnames=("interpret",))
def _yolo_loss_impl(p3, p4, p5, targets, anchors, n3, n4, n5,
                    interpret=False):
    f32 = jnp.float32
    nj = jnp.stack([n3, n4, n5], axis=1).astype(jnp.int32)
    p45 = jnp.concatenate(
        [p4.reshape(_B, 27, 1600), p5.reshape(_B, 27, 400)], axis=2)

    res = pl.pallas_call(
        _yolo_kernel,
        grid=(_B,),
        in_specs=[
            pl.BlockSpec((1, _T, 5), lambda b: (b, 0, 0)),
            pl.BlockSpec(memory_space=pltpu.SMEM),
            pl.BlockSpec((1, 27, 80, 80), lambda b: (b, 0, 0, 0)),
            pl.BlockSpec((1, 27, 2000), lambda b: (b, 0, 0)),
            pl.BlockSpec((1, 3, _TNEG), lambda b: (b, 0, 0)),
        ],
        out_specs=pl.BlockSpec((1, 128), lambda b: (0, 0)),
        out_shape=jax.ShapeDtypeStruct((1, 128), f32),
        scratch_shapes=[pltpu.VMEM((1, 128), f32)],
        compiler_params=pltpu.CompilerParams(
            dimension_semantics=("arbitrary",)),
        interpret=interpret,
    )(targets, anchors, p3, p45, nj)
    return res[0, 0], res[0, 1], res[0, 2], res[0, 3]


def kernel(p3_out, p4_out, p5_out, targets, anchors,
           neg_idx_p3, neg_idx_p4, neg_idx_p5):
    return _yolo_loss_impl(p3_out, p4_out, p5_out, targets, anchors,
                           neg_idx_p3, neg_idx_p4, neg_idx_p5)


# speedup vs baseline: 1.0643x; 1.0643x over previous
import functools

import jax
import jax.numpy as jnp
from jax.experimental import pallas as pl
from jax.experimental.pallas import tpu as pltpu

_NUM_CLASSES = 4
_IMG_SIZE = 640.0
_EPS = 1e-07
_B = 64
_T = 50
_TNEG = 500


def _giou(px, py, pw, ph, tx, ty, tw, th):
    x1n, y1n, x1x, y1x = px - pw / 2, py - ph / 2, px + pw / 2, py + ph / 2
    x2n, y2n, x2x, y2x = tx - tw / 2, ty - th / 2, tx + tw / 2, ty + th / 2
    iw = jnp.clip(jnp.minimum(x1x, x2x) - jnp.maximum(x1n, x2n), 0.0)
    ih = jnp.clip(jnp.minimum(y1x, y2x) - jnp.maximum(y1n, y2n), 0.0)
    inter = iw * ih
    union = pw * ph + tw * th - inter + _EPS
    iou = inter / union
    ew = jnp.maximum(x1x, x2x) - jnp.minimum(x1n, x2n)
    eh = jnp.maximum(y1x, y2x) - jnp.minimum(y1n, y2n)
    enclose = ew * eh + _EPS
    return iou - (enclose - union) / enclose


def _yolo_kernel(tgt_ref, anc_ref, p3_ref, p4_ref, p5_ref, nrc_ref,
                 out_ref, acc_ref):
    f32 = jnp.float32
    bf16 = jnp.bfloat16
    b = pl.program_id(0)
    tt = tgt_ref[0].T
    cls_id = tt[0:1, :].astype(jnp.int32)
    tx, ty, tw, th = tt[1:2, :], tt[2:3, :], tt[3:4, :], tt[4:5, :]
    valid = ((tx >= 0) & (tx <= 1) & (ty >= 0) & (ty <= 1) &
             (tw > 0) & (tw <= 1) & (th > 0) & (th <= 1))
    mask = (valid & (cls_id < _NUM_CLASSES)).astype(f32)
    n_s = jnp.sum(valid.astype(f32)) * 3.0

    def _div(j, w):
        return jnp.floor((j.astype(f32) + 0.5) * (1.0 / w)).astype(jnp.int32)

    nj = nrc_ref[0]

    j3 = nj[0:1, :]
    nr3 = _div(j3, 80)
    nc3 = j3 - nr3 * 80

    def one_scale(pred, h, w, a0, nrow, ncol):
        aw = [anc_ref[a0 + a, 0] / _IMG_SIZE for a in range(3)]
        ah = [anc_ref[a0 + a, 1] / _IMG_SIZE for a in range(3)]

        def ratio(a):
            inter = jnp.minimum(tw, aw[a]) * jnp.minimum(th, ah[a])
            union = tw * th + aw[a] * ah[a] - inter
            return inter / (union + _EPS)
        r0, r1, r2 = ratio(0), ratio(1), ratio(2)
        best = jnp.where(r1 > r0, 1, 0)
        best = jnp.where(r2 > jnp.maximum(r0, r1), 2, best)

        gxi = jnp.clip(jnp.floor(tx * w).astype(jnp.int32), 0, w - 1)
        gyi = jnp.clip(jnp.floor(ty * h).astype(jnp.int32), 0, h - 1)

        p_hi = pred.astype(bf16)
        p_lo = (pred - p_hi.astype(f32)).astype(bf16)
        phl = jnp.concatenate([p_hi, p_lo], axis=0)
        p2 = phl.reshape(54 * h, w)
        ohx = (jax.lax.broadcasted_iota(jnp.int32, (w, _T), 0)
               == gxi).astype(bf16)
        fx = jax.lax.dot(p2, ohx, preferred_element_type=f32)
        fsum = fx[0:27 * h] + fx[27 * h:]
        fa = jnp.where(best == 0, fsum[0:9 * h],
                       jnp.where(best == 1, fsum[9 * h:18 * h],
                                 fsum[18 * h:27 * h]))
        f9 = fa.reshape(9, h, _T)
        ohy = (jax.lax.broadcasted_iota(jnp.int32, (1, h, _T), 1)
               == gyi[:, None, :]).astype(f32)
        z = jnp.sum(f9 * ohy, axis=1)

        aw_sel = jnp.where(best == 0, aw[0], jnp.where(best == 1, aw[1], aw[2]))
        ah_sel = jnp.where(best == 0, ah[0], jnp.where(best == 1, ah[1], ah[2]))

        px = jax.nn.sigmoid(z[0:1]) + gxi.astype(f32) / w
        py = jax.nn.sigmoid(z[1:2]) + gyi.astype(f32) / h
        pw_ = jnp.exp(z[2:3]) * aw_sel
        ph_ = jnp.exp(z[3:4]) * ah_sel
        giou = _giou(px, py, pw_, ph_, tx, ty, tw, th)

        box_s = jnp.sum(mask * (1.0 - giou))
        zcl = z[5:9]
        ohc = (jax.lax.broadcasted_iota(jnp.int32, (_NUM_CLASSES, _T), 0)
               == cls_id).astype(f32)
        bce = jnp.sum(jax.nn.softplus(zcl) - ohc * zcl, axis=0,
                      keepdims=True) / _NUM_CLASSES
        cls_s = jnp.sum(mask * bce)
        zc = z[4:5]
        conf_t = jnp.clip(giou, 0.0, 1.0)
        conf_s = jnp.sum(mask * (jax.nn.softplus(zc) - conf_t * zc))

        sp = jax.nn.softplus(
            jnp.concatenate([pred[4], pred[13], pred[22]], axis=0))
        ohr = (jax.lax.broadcasted_iota(jnp.int32, (3 * h, _TNEG), 0)
               == nrow).astype(bf16)
        ohcol = (jax.lax.broadcasted_iota(jnp.int32, (128, _TNEG), 0)
                 == ncol).astype(bf16)
        counts = jax.lax.dot_general(ohr, ohcol, (((1,), (1,)), ((), ())),
                                     preferred_element_type=f32)
        neg_s = jnp.sum(sp * counts[:, 0:w])
        return box_s, cls_s, conf_s, neg_s

    def flat_scale(pred, h, w, a0, nrow, ncol, nrows):
        hw = h * w
        aw = [anc_ref[a0 + a, 0] / _IMG_SIZE for a in range(3)]
        ah = [anc_ref[a0 + a, 1] / _IMG_SIZE for a in range(3)]

        def ratio(a):
            inter = jnp.minimum(tw, aw[a]) * jnp.minimum(th, ah[a])
            union = tw * th + aw[a] * ah[a] - inter
            return inter / (union + _EPS)
        r0, r1, r2 = ratio(0), ratio(1), ratio(2)
        best = jnp.where(r1 > r0, 1, 0)
        best = jnp.where(r2 > jnp.maximum(r0, r1), 2, best)

        gxi = jnp.clip(jnp.floor(tx * w).astype(jnp.int32), 0, w - 1)
        gyi = jnp.clip(jnp.floor(ty * h).astype(jnp.int32), 0, h - 1)
        s = gyi * w + gxi

        p_hi = pred.astype(bf16)
        p_lo = (pred - p_hi.astype(f32)).astype(bf16)
        phl = jnp.concatenate([p_hi, p_lo], axis=0)
        ohs = (jax.lax.broadcasted_iota(jnp.int32, (hw, _T), 0)
               == s).astype(bf16)
        fx = jax.lax.dot(phl, ohs, preferred_element_type=f32)
        fsum = fx[0:27] + fx[27:]
        z = jnp.where(best == 0, fsum[0:9],
                      jnp.where(best == 1, fsum[9:18],
                                fsum[18:27]))

        aw_sel = jnp.where(best == 0, aw[0], jnp.where(best == 1, aw[1], aw[2]))
        ah_sel = jnp.where(best == 0, ah[0], jnp.where(best == 1, ah[1], ah[2]))
        px = jax.nn.sigmoid(z[0:1]) + gxi.astype(f32) / w
        py = jax.nn.sigmoid(z[1:2]) + gyi.astype(f32) / h
        pw_ = jnp.exp(z[2:3]) * aw_sel
        ph_ = jnp.exp(z[3:4]) * ah_sel
        giou = _giou(px, py, pw_, ph_, tx, ty, tw, th)

        box_s = jnp.sum(mask * (1.0 - giou))
        zcl = z[5:9]
        ohc = (jax.lax.broadcasted_iota(jnp.int32, (_NUM_CLASSES, _T), 0)
               == cls_id).astype(f32)
        bce = jnp.sum(jax.nn.softplus(zcl) - ohc * zcl, axis=0,
                      keepdims=True) / _NUM_CLASSES
        cls_s = jnp.sum(mask * bce)
        zc = z[4:5]
        conf_t = jnp.clip(giou, 0.0, 1.0)
        conf_s = jnp.sum(mask * (jax.nn.softplus(zc) - conf_t * zc))

        hwpad = ((hw + 127) // 128) * 128
        sp0 = jax.nn.softplus(
            jnp.concatenate([pred[4:5], pred[13:14], pred[22:23]], axis=0))
        if hwpad > hw:
            sp0 = jnp.concatenate(
                [sp0, jnp.zeros((3, hwpad - hw), f32)], axis=1)
        sp = sp0.reshape(nrows, 128)
        ohr = (jax.lax.broadcasted_iota(jnp.int32, (nrows, _TNEG), 0)
               == nrow).astype(bf16)
        ohcol = (jax.lax.broadcasted_iota(jnp.int32, (128, _TNEG), 0)
                 == ncol).astype(bf16)
        counts = jax.lax.dot_general(ohr, ohcol, (((1,), (1,)), ((), ())),
                                     preferred_element_type=f32)
        neg_s = jnp.sum(sp * counts)
        return box_s, cls_s, conf_s, neg_s

    def padded_rc(si, hw):
        hwpad = ((hw + 127) // 128) * 128
        j = nj[si:si + 1, :]
        a = (j >= hw).astype(jnp.int32) + (j >= 2 * hw).astype(jnp.int32)
        jp = a * hwpad + (j - a * hw)
        return jp // 128, jp - (jp // 128) * 128

    nr4, nc4 = padded_rc(1, 1600)
    nr5, nc5 = padded_rc(2, 400)
    b3 = one_scale(p3_ref[0], 80, 80, 0, nr3, nc3)
    b4 = flat_scale(p4_ref[0], 40, 40, 3, nr4, nc4, 39)
    b5 = flat_scale(p5_ref[0], 20, 20, 6, nr5, nc5, 12)
    box_s = b3[0] + b4[0] + b5[0]
    cls_s = b3[1] + b4[1] + b5[1]
    conf_s = b3[2] + b4[2] + b5[2]
    neg_s = b3[3] + b4[3] + b5[3]

    lanes = jax.lax.broadcasted_iota(jnp.int32, (1, 128), 1)
    vec = (jnp.where(lanes == 0, box_s, 0.0) +
           jnp.where(lanes == 1, cls_s, 0.0) +
           jnp.where(lanes == 2, conf_s, 0.0) +
           jnp.where(lanes == 3, n_s, 0.0) +
           jnp.where(lanes == 4, neg_s, 0.0))

    @pl.when(b == 0)
    def _():
        acc_ref[...] = vec

    @pl.when(b > 0)
    def _():
        acc_ref[...] += vec

    @pl.when(b == _B - 1)
    def _():
        v = acc_ref[...]

        def lane(k):
            return jnp.sum(jnp.where(lanes == k, v, 0.0))
        box, cls_, conf = lane(0), lane(1), lane(2)
        n, neg = lane(3), lane(4)
        denom = jnp.maximum(n, 1.0)
        pos = n > 0
        box = jnp.where(pos, box / denom, box)
        cls_ = jnp.where(pos, cls_ / denom, cls_)
        conf = jnp.where(pos, conf / denom, conf)
        conf = conf + neg * (0.5 / (_TNEG * float(_B * _TNEG)))
        total = box + cls_ + conf
        out_ref[...] = (jnp.where(lanes == 0, total, 0.0) +
                        jnp.where(lanes == 1, box, 0.0) +
                        jnp.where(lanes == 2, cls_, 0.0) +
                        jnp.where(lanes == 3, conf, 0.0))


@functools.partial(jax.jit, static_argnames=("interpret",))
def _yolo_loss_impl(p3, p4, p5, targets, anchors, n3, n4, n5,
                    interpret=False):
    f32 = jnp.float32
    nj = jnp.stack([n3, n4, n5], axis=1).astype(jnp.int32)
    p4f = p4.reshape(_B, 27, 1600)
    p5f = p5.reshape(_B, 27, 400)

    res = pl.pallas_call(
        _yolo_kernel,
        grid=(_B,),
        in_specs=[
            pl.BlockSpec((1, _T, 5), lambda b: (b, 0, 0)),
            pl.BlockSpec(memory_space=pltpu.SMEM),
            pl.BlockSpec((1, 27, 80, 80), lambda b: (b, 0, 0, 0)),
            pl.BlockSpec((1, 27, 1600), lambda b: (b, 0, 0)),
            pl.BlockSpec((1, 27, 400), lambda b: (b, 0, 0)),
            pl.BlockSpec((1, 3, _TNEG), lambda b: (b, 0, 0)),
        ],
        out_specs=pl.BlockSpec((1, 128), lambda b: (0, 0)),
        out_shape=jax.ShapeDtypeStruct((1, 128), f32),
        scratch_shapes=[pltpu.VMEM((1, 128), f32)],
        compiler_params=pltpu.CompilerParams(
            dimension_semantics=("arbitrary",)),
        interpret=interpret,
    )(targets, anchors, p3, p4f, p5f, nj)
    return res[0, 0], res[0, 1], res[0, 2], res[0, 3]


def kernel(p3_out, p4_out, p5_out, targets, anchors,
           neg_idx_p3, neg_idx_p4, neg_idx_p5):
    return _yolo_loss_impl(p3_out, p4_out, p5_out, targets, anchors,
                           neg_idx_p3, neg_idx_p4, neg_idx_p5)
